# baseline (speedup 1.0000x reference)
import functools

import jax
import jax.numpy as jnp
from jax import lax
from jax.experimental import pallas as pl
from jax.experimental.pallas import tpu as pltpu
from jax.experimental.pallas import tpu_sc as plsc

NC = 2
NS = 16
NW = NC * NS
L = 16
CHUNK = 128


def _mirt_body(bpw, nchunk, nvec, n_stu, n_exer,
               stu_hbm, exer_hbm, th_hbm, ax_hbm,
               out_hbm,
               stu_v, exer_v, i1_v, i2_v, i3_v,
               t0_v, t1_v, a0_v, a1_v, b_v, out_v, sem):
    wid = lax.axis_index("s") * NC + lax.axis_index("c")
    base = wid * bpw

    pltpu.sync_copy(stu_hbm.at[pl.ds(base, bpw)], stu_v)
    pltpu.sync_copy(exer_hbm.at[pl.ds(base, bpw)], exer_v)

    def idx_body(j, carry):
        sl = pl.ds(j * L, L)
        e = exer_v[sl]
        i1_v[sl] = stu_v[sl] + n_stu
        i2_v[sl] = e + n_exer
        i3_v[sl] = e + 2 * n_exer
        return carry

    lax.fori_loop(0, nvec, idx_body, 0)

    copies = []
    for c in range(nchunk):
        sl = pl.ds(c * CHUNK, CHUNK)
        copies.append(pltpu.async_copy(th_hbm.at[stu_v.at[sl]], t0_v.at[sl], sem))
        copies.append(pltpu.async_copy(th_hbm.at[i1_v.at[sl]], t1_v.at[sl], sem))
        copies.append(pltpu.async_copy(ax_hbm.at[exer_v.at[sl]], a0_v.at[sl], sem))
        copies.append(pltpu.async_copy(ax_hbm.at[i2_v.at[sl]], a1_v.at[sl], sem))
        copies.append(pltpu.async_copy(ax_hbm.at[i3_v.at[sl]], b_v.at[sl], sem))
    for cp in copies:
        cp.wait()

    def vec_body(j, carry):
        sl = pl.ds(j * L, L)
        t0 = t0_v[sl]
        t1 = t1_v[sl]
        a0 = a0_v[sl]
        a1 = a1_v[sl]
        b = b_v[sl]
        pred = a0 * (t0 - b) + a1 * (t1 - b)
        out_v[sl] = 1.0 / (1.0 + jnp.exp(-pred))
        return carry

    lax.fori_loop(0, nvec, vec_body, 0)

    pltpu.sync_copy(out_v, out_hbm.at[pl.ds(base, bpw)])


def _build(batch, n_stu, n_exer):
    bpw = batch // NW
    nchunk = bpw // CHUNK
    nvec = bpw // L
    mesh = plsc.VectorSubcoreMesh(core_axis_name="c", subcore_axis_name="s")
    idx = pltpu.VMEM((bpw,), jnp.int32)
    val = pltpu.VMEM((bpw,), jnp.float32)
    return functools.partial(
        pl.kernel,
        out_type=jax.ShapeDtypeStruct((batch,), jnp.float32),
        mesh=mesh,
        scratch_types=[idx, idx, idx, idx, idx,
                       val, val, val, val, val, val,
                       pltpu.SemaphoreType.DMA],
    )(functools.partial(_mirt_body, bpw, nchunk, nvec, n_stu, n_exer))


def kernel(stu_id, exer_id, theta_table, alpha_table, beta_table):
    batch = stu_id.shape[0]
    stu = stu_id.astype(jnp.int32)
    exer = exer_id.astype(jnp.int32)
    th_soa = jnp.ravel(theta_table.T)
    ax_soa = jnp.ravel(jnp.concatenate([alpha_table, beta_table], axis=1).T)
    return _build(batch, theta_table.shape[0], alpha_table.shape[0])(
        stu, exer, th_soa, ax_soa)


# speedup vs baseline: 1.0088x; 1.0088x over previous
import functools

import jax
import jax.numpy as jnp
from jax import lax
from jax.experimental import pallas as pl
from jax.experimental.pallas import tpu as pltpu
from jax.experimental.pallas import tpu_sc as plsc

NC = 2
NS = 16
NW = NC * NS
L = 16
CHUNK = 128


def _ex_body(bpw, nchunk, nvec, n_exer,
             exer_hbm, ax_hbm,
             a0_hbm, a1_hbm, b_hbm,
             exer_v, i2_v, i3_v, a0_v, a1_v, b_v, sem):
    wid = lax.axis_index("s") * NC + lax.axis_index("c")
    base = wid * bpw

    pltpu.sync_copy(exer_hbm.at[pl.ds(base, bpw)], exer_v)

    def idx_body(j, carry):
        sl = pl.ds(j * L, L)
        e = exer_v[sl]
        i2_v[sl] = e + n_exer
        i3_v[sl] = e + 2 * n_exer
        return carry

    lax.fori_loop(0, nvec, idx_body, 0)

    copies = []
    for c in range(nchunk):
        sl = pl.ds(c * CHUNK, CHUNK)
        copies.append(pltpu.async_copy(ax_hbm.at[exer_v.at[sl]], a0_v.at[sl], sem))
        copies.append(pltpu.async_copy(ax_hbm.at[i2_v.at[sl]], a1_v.at[sl], sem))
        copies.append(pltpu.async_copy(ax_hbm.at[i3_v.at[sl]], b_v.at[sl], sem))
    for cp in copies:
        cp.wait()

    pltpu.sync_copy(a0_v, a0_hbm.at[pl.ds(base, bpw)])
    pltpu.sync_copy(a1_v, a1_hbm.at[pl.ds(base, bpw)])
    pltpu.sync_copy(b_v, b_hbm.at[pl.ds(base, bpw)])


def _th_body(bpw, nchunk, nvec, n_stu,
             stu_hbm, th_hbm, a0_hbm, a1_hbm, b_hbm,
             out_hbm,
             stu_v, i1_v, t0_v, t1_v, a0_v, a1_v, b_v, out_v, sem):
    wid = lax.axis_index("s") * NC + lax.axis_index("c")
    base = wid * bpw

    pltpu.sync_copy(stu_hbm.at[pl.ds(base, bpw)], stu_v)
    pltpu.sync_copy(a0_hbm.at[pl.ds(base, bpw)], a0_v)
    pltpu.sync_copy(a1_hbm.at[pl.ds(base, bpw)], a1_v)
    pltpu.sync_copy(b_hbm.at[pl.ds(base, bpw)], b_v)

    def idx_body(j, carry):
        sl = pl.ds(j * L, L)
        i1_v[sl] = stu_v[sl] + n_stu
        return carry

    lax.fori_loop(0, nvec, idx_body, 0)

    copies = []
    for c in range(nchunk):
        sl = pl.ds(c * CHUNK, CHUNK)
        copies.append(pltpu.async_copy(th_hbm.at[stu_v.at[sl]], t0_v.at[sl], sem))
        copies.append(pltpu.async_copy(th_hbm.at[i1_v.at[sl]], t1_v.at[sl], sem))
    for cp in copies:
        cp.wait()

    def vec_body(j, carry):
        sl = pl.ds(j * L, L)
        t0 = t0_v[sl]
        t1 = t1_v[sl]
        a0 = a0_v[sl]
        a1 = a1_v[sl]
        b = b_v[sl]
        pred = a0 * (t0 - b) + a1 * (t1 - b)
        out_v[sl] = 1.0 / (1.0 + jnp.exp(-pred))
        return carry

    lax.fori_loop(0, nvec, vec_body, 0)

    pltpu.sync_copy(out_v, out_hbm.at[pl.ds(base, bpw)])


def _build_ex(batch, n_exer):
    bpw = batch // NW
    nchunk = bpw // CHUNK
    nvec = bpw // L
    mesh = plsc.VectorSubcoreMesh(core_axis_name="c", subcore_axis_name="s")
    idx = pltpu.VMEM((bpw,), jnp.int32)
    val = pltpu.VMEM((bpw,), jnp.float32)
    o = jax.ShapeDtypeStruct((batch,), jnp.float32)
    return functools.partial(
        pl.kernel,
        out_type=(o, o, o),
        mesh=mesh,
        scratch_types=[idx, idx, idx, val, val, val,
                       pltpu.SemaphoreType.DMA],
    )(functools.partial(_ex_body, bpw, nchunk, nvec, n_exer))


def _build_th(batch, n_stu):
    bpw = batch // NW
    nchunk = bpw // CHUNK
    nvec = bpw // L
    mesh = plsc.VectorSubcoreMesh(core_axis_name="c", subcore_axis_name="s")
    idx = pltpu.VMEM((bpw,), jnp.int32)
    val = pltpu.VMEM((bpw,), jnp.float32)
    return functools.partial(
        pl.kernel,
        out_type=jax.ShapeDtypeStruct((batch,), jnp.float32),
        mesh=mesh,
        scratch_types=[idx, idx, val, val, val, val, val, val,
                       pltpu.SemaphoreType.DMA],
    )(functools.partial(_th_body, bpw, nchunk, nvec, n_stu))


def kernel(stu_id, exer_id, theta_table, alpha_table, beta_table):
    batch = stu_id.shape[0]
    stu = stu_id.astype(jnp.int32)
    exer = exer_id.astype(jnp.int32)
    th_soa = jnp.ravel(theta_table.T)
    ax_soa = jnp.ravel(jnp.concatenate([alpha_table, beta_table], axis=1).T)
    a0g, a1g, bg = _build_ex(batch, alpha_table.shape[0])(exer, ax_soa)
    return _build_th(batch, theta_table.shape[0])(stu, th_soa, a0g, a1g, bg)
